# baseline (speedup 1.0000x reference)
import functools

import jax
import jax.numpy as jnp
from jax import lax
from jax.experimental import pallas as pl
from jax.experimental.pallas import tpu as pltpu
from jax.experimental.pallas import tpu_sc as plsc

_D = 64
_L = 128
_P = 129
_U = 2
_NBUF = 2


def _build(B, H):
    NW = 32
    nblk = H * (B // _L)
    blk_per_w = nblk // NW
    unit_per_w = blk_per_w // _U
    rblk = B // _L

    mesh = plsc.VectorSubcoreMesh(core_axis_name="c", subcore_axis_name="s")

    @functools.partial(
        pl.kernel,
        mesh=mesh,
        out_type=jax.ShapeDtypeStruct((H, _D // 8, rblk, 8, _L), jnp.float32),
        scratch_types=[
            pltpu.VMEM((_NBUF, _U, _L), jnp.int32),
            pltpu.VMEM((_NBUF, _U * _L, _D), jnp.float32),
            pltpu.VMEM((_NBUF, _D // 8, _U, 8, _P), jnp.float32),
            pltpu.SemaphoreType.DMA((_NBUF,)),
            pltpu.SemaphoreType.DMA((_NBUF,)),
            pltpu.SemaphoreType.DMA((_NBUF,)),
        ],
        compiler_params=pltpu.CompilerParams(
            use_tc_tiling_on_sc=False, needs_layout_passes=False
        ),
    )
    def k(ctx_hbm, table_hbm, out_hbm, idx_v, rows_v, tr_v, sem_i, sem_g, sem_o):
        wid = lax.axis_index("s") * 2 + lax.axis_index("c")
        u0 = wid * unit_per_w
        lanes = lax.iota(jnp.int32, 16)
        chi_vecs = [(lanes + 16 * g) // 8 for g in range(_D // 16)]
        clo_vecs = [lax.rem(lanes + 16 * g, 8) for g in range(_D // 16)]
        tsplats = [jnp.full((16,), t, dtype=jnp.int32) for t in range(_U)]

        def gathers(b, start):
            for t in range(_U):
                cp = pltpu.make_async_copy(
                    table_hbm.at[idx_v.at[b, t]],
                    rows_v.at[b, pl.ds(_L * t, _L)],
                    sem_g.at[b],
                )
                if start:
                    cp.start()
                else:
                    cp.wait()

        def out_copy(u, b, start):
            blk = u * _U
            h = lax.div(blk, rblk)
            r = lax.rem(blk, rblk)
            cp = pltpu.make_async_copy(
                tr_v.at[b, :, :, :, pl.ds(0, _L)],
                out_hbm.at[h, :, pl.ds(r, _U)],
                sem_o.at[b],
            )
            if start:
                cp.start()
            else:
                cp.wait()

        def body(i, carry):
            for b in range(_NBUF):
                u = u0 + i * _NBUF + b
                gathers(b, start=False)

                @pl.when(i > 0)
                def _wait_prev_out():
                    out_copy(u - _NBUF, b, start=False)

                @pl.when(i * _NBUF + b + _NBUF < unit_per_w)
                def _prefetch_idx():
                    pltpu.async_copy(
                        ctx_hbm.at[pl.ds((u + _NBUF) * _U, _U)],
                        idx_v.at[b],
                        sem_i.at[b],
                    )

                def tgrp(rg, carry2):
                    for t in range(_U):
                        for rl in range(4):
                            rp = 4 * rg + rl
                            rsplat = jnp.full((16,), rp, dtype=jnp.int32)
                            for g in range(_D // 16):
                                v = rows_v[b, _L * t + rp, pl.ds(16 * g, 16)]
                                plsc.store_scatter(
                                    tr_v.at[b],
                                    [chi_vecs[g], tsplats[t], clo_vecs[g], rsplat],
                                    v,
                                )
                    return carry2

                lax.fori_loop(0, _L // 4, tgrp, 0)

                out_copy(u, b, start=True)

                @pl.when(i * _NBUF + b + _NBUF < unit_per_w)
                def _next_gather():
                    pltpu.make_async_copy(
                        ctx_hbm.at[pl.ds((u + _NBUF) * _U, _U)],
                        idx_v.at[b],
                        sem_i.at[b],
                    ).wait()
                    gathers(b, start=True)

            return carry

        for b in range(_NBUF):
            pltpu.sync_copy(ctx_hbm.at[pl.ds((u0 + b) * _U, _U)], idx_v.at[b])
            gathers(b, start=True)

        lax.fori_loop(0, unit_per_w // _NBUF, body, 0)

        for b in range(_NBUF):
            out_copy(u0 + unit_per_w - _NBUF + b, b, start=False)

    return k


def kernel(context, table):
    B, H = context.shape
    ctx_cols = context.T.reshape(H * (B // _L), _L)
    out5 = _build(B, H)(ctx_cols, table)
    return out5.transpose(2, 4, 0, 1, 3).reshape(B, H, _D)


# speedup vs baseline: 1.0326x; 1.0326x over previous
import functools

import jax
import jax.numpy as jnp
from jax import lax
from jax.experimental import pallas as pl
from jax.experimental.pallas import tpu as pltpu
from jax.experimental.pallas import tpu_sc as plsc

_D = 64
_L = 128
_P = 129
_NBUF = 2


def _build(B, H):
    NW = 32
    nblk = H * (B // _L)
    blk_per_w = nblk // NW
    rblk = B // _L

    mesh = plsc.VectorSubcoreMesh(core_axis_name="c", subcore_axis_name="s")

    @functools.partial(
        pl.kernel,
        mesh=mesh,
        out_type=jax.ShapeDtypeStruct((H, _D // 8, rblk, 8, _L), jnp.float32),
        scratch_types=[
            pltpu.VMEM((_NBUF, _L), jnp.int32),
            pltpu.VMEM((_NBUF, _L, _D), jnp.float32),
            pltpu.VMEM((_NBUF, _D // 8, 8, _P), jnp.float32),
            pltpu.SemaphoreType.DMA((_NBUF,)),
            pltpu.SemaphoreType.DMA((_NBUF,)),
            pltpu.SemaphoreType.DMA((_NBUF,)),
        ],
        compiler_params=pltpu.CompilerParams(
            use_tc_tiling_on_sc=False, needs_layout_passes=False
        ),
    )
    def k(ctx_hbm, table_hbm, out_hbm, idx_v, rows_v, tr_v, sem_i, sem_g, sem_o):
        def ctx_row(j):
            h = lax.div(j, rblk)
            r = lax.rem(j, rblk)
            return ctx_hbm.at[lax.div(h, 8), r, lax.rem(h, 8)]
        wid = lax.axis_index("s") * 2 + lax.axis_index("c")
        blk0 = wid * blk_per_w
        lanes = lax.iota(jnp.int32, 16)
        cvecs = [lanes + 16 * g for g in range(_D // 16)]
        chi_vecs = [(lanes + 16 * g) // 8 for g in range(_D // 16)]
        clo_vecs = [lax.rem(lanes + 16 * g, 8) for g in range(_D // 16)]

        def out_copies(j, b, start):
            h = lax.div(j, rblk)
            r = lax.rem(j, rblk)
            cp = pltpu.make_async_copy(
                tr_v.at[b, :, :, pl.ds(0, _L)],
                out_hbm.at[h, :, r],
                sem_o.at[b],
            )
            if start:
                cp.start()
            else:
                cp.wait()

        def body(i, carry):
            for b in range(_NBUF):
                j = blk0 + i * _NBUF + b
                pltpu.make_async_copy(
                    table_hbm.at[idx_v.at[b]], rows_v.at[b], sem_g.at[b]
                ).wait()

                @pl.when(i > 0)
                def _wait_prev_out():
                    out_copies(j - _NBUF, b, start=False)

                @pl.when(i * _NBUF + b + _NBUF < blk_per_w)
                def _prefetch_idx():
                    pltpu.async_copy(ctx_row(j + _NBUF), idx_v.at[b], sem_i.at[b])

                def tgrp(rg, carry2):
                    for rl in range(8):
                        rsplat = jnp.full((16,), 8 * rg + rl, dtype=jnp.int32)
                        for g in range(_D // 16):
                            v = rows_v[b, 8 * rg + rl, pl.ds(16 * g, 16)]
                            plsc.store_scatter(
                                tr_v.at[b], [chi_vecs[g], clo_vecs[g], rsplat], v
                            )
                    return carry2

                lax.fori_loop(0, _L // 8, tgrp, 0)

                out_copies(j, b, start=True)

                @pl.when(i * _NBUF + b + _NBUF < blk_per_w)
                def _next_gather():
                    pltpu.make_async_copy(
                        ctx_row(j + _NBUF), idx_v.at[b], sem_i.at[b]
                    ).wait()
                    pltpu.async_copy(
                        table_hbm.at[idx_v.at[b]], rows_v.at[b], sem_g.at[b]
                    )

            return carry

        for b in range(_NBUF):
            pltpu.sync_copy(ctx_row(blk0 + b), idx_v.at[b])
            pltpu.async_copy(table_hbm.at[idx_v.at[b]], rows_v.at[b], sem_g.at[b])

        lax.fori_loop(0, blk_per_w // _NBUF, body, 0)

        for b in range(_NBUF):
            out_copies(blk0 + blk_per_w - _NBUF + b, b, start=False)

    return k


def kernel(context, table):
    B, H = context.shape
    ctx_phys = context.reshape(B // _L, _L, H // 8, 8).transpose(2, 0, 3, 1)
    out5 = _build(B, H)(ctx_phys, table)
    return out5.transpose(2, 4, 0, 1, 3).reshape(B, H, _D)


# speedup vs baseline: 1.0351x; 1.0024x over previous
import functools

import jax
import jax.numpy as jnp
from jax import lax
from jax.experimental import pallas as pl
from jax.experimental.pallas import tpu as pltpu
from jax.experimental.pallas import tpu_sc as plsc

_D = 64
_L = 128
_P = 129
_NBUF = 2


def _build(B, H):
    NW = 32
    nblk = H * (B // _L)
    blk_per_w = nblk // NW
    rblk = B // _L

    mesh = plsc.VectorSubcoreMesh(core_axis_name="c", subcore_axis_name="s")

    @functools.partial(
        pl.kernel,
        mesh=mesh,
        out_type=jax.ShapeDtypeStruct((H, _D // 8, rblk, 8, _L), jnp.float32),
        scratch_types=[
            pltpu.VMEM((_NBUF, _L), jnp.int32),
            pltpu.VMEM((_NBUF, _L, _D), jnp.float32),
            pltpu.VMEM((_NBUF, _D // 8, 8, _P), jnp.float32),
            pltpu.SemaphoreType.DMA((_NBUF,)),
            pltpu.SemaphoreType.DMA((_NBUF,)),
            pltpu.SemaphoreType.DMA((_NBUF,)),
        ],
        compiler_params=pltpu.CompilerParams(
            use_tc_tiling_on_sc=False, needs_layout_passes=False
        ),
    )
    def k(ctx_hbm, table_hbm, out_hbm, idx_v, rows_v, tr_v, sem_i, sem_g, sem_o):
        def ctx_row(j):
            h = lax.div(j, rblk)
            r = lax.rem(j, rblk)
            return ctx_hbm.at[lax.div(h, 8), r, lax.rem(h, 8)]
        wid = lax.axis_index("s") * 2 + lax.axis_index("c")
        blk0 = wid * blk_per_w
        lanes = lax.iota(jnp.int32, 16)
        chi_vecs = [(lanes + 16 * g) // 8 for g in range(_D // 16)]
        clo_vecs = [lax.rem(lanes + 16 * g, 8) for g in range(_D // 16)]

        def out_copies(j, b, start):
            h = lax.div(j, rblk)
            r = lax.rem(j, rblk)
            cp = pltpu.make_async_copy(
                tr_v.at[b, :, :, pl.ds(0, _L)],
                out_hbm.at[h, :, r],
                sem_o.at[b],
            )
            if start:
                cp.start()
            else:
                cp.wait()

        def body(i, carry):
            for b in range(_NBUF):
                j = blk0 + i * _NBUF + b
                pltpu.make_async_copy(
                    table_hbm.at[idx_v.at[b]], rows_v.at[b], sem_g.at[b]
                ).wait()

                @pl.when(i > 0)
                def _wait_prev_out():
                    out_copies(j - _NBUF, b, start=False)

                @pl.when(i * _NBUF + b + _NBUF < blk_per_w)
                def _prefetch_idx():
                    pltpu.async_copy(ctx_row(j + _NBUF), idx_v.at[b], sem_i.at[b])

                def tgrp(rg, carry2):
                    for rl in range(8):
                        rsplat = jnp.full((16,), 8 * rg + rl, dtype=jnp.int32)
                        for g in range(_D // 16):
                            v = rows_v[b, 8 * rg + rl, pl.ds(16 * g, 16)]
                            plsc.store_scatter(
                                tr_v.at[b], [chi_vecs[g], clo_vecs[g], rsplat], v
                            )
                    return carry2

                lax.fori_loop(0, _L // 8, tgrp, 0)

                out_copies(j, b, start=True)

                @pl.when(i * _NBUF + b + _NBUF < blk_per_w)
                def _next_gather():
                    pltpu.make_async_copy(
                        ctx_row(j + _NBUF), idx_v.at[b], sem_i.at[b]
                    ).wait()
                    pltpu.async_copy(
                        table_hbm.at[idx_v.at[b]], rows_v.at[b], sem_g.at[b]
                    )

            return carry

        for b in range(_NBUF):
            pltpu.sync_copy(ctx_row(blk0 + b), idx_v.at[b])
            pltpu.async_copy(table_hbm.at[idx_v.at[b]], rows_v.at[b], sem_g.at[b])

        lax.fori_loop(0, blk_per_w // _NBUF, body, 0)

        for b in range(_NBUF):
            out_copies(blk0 + blk_per_w - _NBUF + b, b, start=False)

    return k


def kernel(context, table):
    B, H = context.shape
    ctx_phys = context.reshape(B // _L, _L, H // 8, 8).transpose(2, 0, 3, 1)
    out5 = _build(B, H)(ctx_phys, table)
    return out5.transpose(2, 4, 0, 1, 3).reshape(B, H, _D)
